# baseline (speedup 1.0000x reference)
import functools

import jax
import jax.numpy as jnp
from jax import lax
from jax.experimental import pallas as pl
from jax.experimental.pallas import tpu as pltpu
from jax.experimental.pallas import tpu_sc as plsc

_NC = 2
_NS = 16
_NW = _NC * _NS


def _segment_sum_sc(vals, src3, dst3, nbuf):
    n, d = vals.shape
    n_chunks, chunk = src3.shape[1], src3.shape[2]
    assert n_chunks % nbuf == 0
    n_row_blocks = n // chunk
    mesh = plsc.VectorSubcoreMesh(core_axis_name="c", subcore_axis_name="s")

    @functools.partial(
        pl.kernel,
        out_type=jax.ShapeDtypeStruct((_NC, n, d), jnp.float32),
        mesh=mesh,
        compiler_params=pltpu.CompilerParams(use_tc_tiling_on_sc=False),
        scratch_types=[
            pltpu.VMEM((n_chunks, chunk), jnp.int32),
            pltpu.VMEM((nbuf, chunk), jnp.int32),
            pltpu.VMEM((nbuf, chunk, d), jnp.float32),
            pltpu.VMEM_SHARED((n, d), jnp.float32),
        ] + [pltpu.SemaphoreType.DMA] * (2 * nbuf),
    )
    def seg_kernel(vals_hbm, src_hbm, dst_hbm, out_hbm,
                   src_v, dstr_v, rows_v, acc, *sems):
        gsem = sems[:nbuf]
        dsem = sems[nbuf:]
        cid = lax.axis_index("c")
        sid = lax.axis_index("s")
        wid = sid * _NC + cid

        pltpu.async_copy(src_hbm.at[wid], src_v, gsem[0])

        zvec = jnp.zeros((16,), jnp.float32)

        @pl.loop(0, chunk)
        def _(r):
            @pl.loop(0, d // 16)
            def _(c):
                rows_v.at[0, r, pl.ds(c * 16, 16)][...] = zvec

        @pl.loop(sid, n_row_blocks, step=_NS)
        def _(b):
            pltpu.async_copy(rows_v.at[0], acc.at[pl.ds(b * chunk, chunk)],
                             dsem[0])

        @pl.loop(sid, n_row_blocks, step=_NS)
        def _(b):
            pltpu.make_async_copy(
                rows_v.at[0], acc.at[pl.ds(b * chunk, chunk)], dsem[0]).wait()

        pltpu.make_async_copy(src_hbm.at[wid], src_v, gsem[0]).wait()

        plsc.subcore_barrier()

        for b in range(nbuf):
            pltpu.async_copy(dst_hbm.at[wid, b], dstr_v.at[b], dsem[b])
            pltpu.async_copy(vals_hbm.at[src_v.at[b]], rows_v.at[b], gsem[b])

        @pl.loop(0, n_chunks - nbuf, step=nbuf)
        def _(i0):
            for b in range(nbuf):
                i = i0 + b
                pltpu.make_async_copy(
                    vals_hbm.at[src_v.at[i]], rows_v.at[b], gsem[b]).wait()
                pltpu.make_async_copy(
                    dst_hbm.at[wid, i], dstr_v.at[b], dsem[b]).wait()
                pltpu.sync_copy(rows_v.at[b], acc.at[dstr_v.at[b]], add=True)
                pltpu.async_copy(
                    dst_hbm.at[wid, i + nbuf], dstr_v.at[b], dsem[b])
                pltpu.async_copy(
                    vals_hbm.at[src_v.at[i + nbuf]], rows_v.at[b], gsem[b])

        for b in range(nbuf):
            i = n_chunks - nbuf + b
            pltpu.make_async_copy(
                vals_hbm.at[src_v.at[i]], rows_v.at[b], gsem[b]).wait()
            pltpu.make_async_copy(
                dst_hbm.at[wid, i], dstr_v.at[b], dsem[b]).wait()
            pltpu.sync_copy(rows_v.at[b], acc.at[dstr_v.at[b]], add=True)

        plsc.subcore_barrier()

        @pl.loop(sid, n_row_blocks, step=_NS)
        def _(b):
            pltpu.async_copy(acc.at[pl.ds(b * chunk, chunk)],
                             out_hbm.at[cid, pl.ds(b * chunk, chunk)],
                             gsem[0])

        @pl.loop(sid, n_row_blocks, step=_NS)
        def _(b):
            pltpu.make_async_copy(
                acc.at[pl.ds(b * chunk, chunk)],
                out_hbm.at[cid, pl.ds(b * chunk, chunk)], gsem[0]).wait()

    return seg_kernel(vals, src3, dst3)


def _segment_sum_sc_colsplit(vals2, src3, dst3, nbuf):
    _, n, dh = vals2.shape
    n_chunks, chunk = src3.shape[1], src3.shape[2]
    assert n_chunks % nbuf == 0
    n_row_blocks = n // chunk
    mesh = plsc.VectorSubcoreMesh(core_axis_name="c", subcore_axis_name="s")

    @functools.partial(
        pl.kernel,
        out_type=jax.ShapeDtypeStruct((_NC, n, dh), jnp.float32),
        mesh=mesh,
        compiler_params=pltpu.CompilerParams(use_tc_tiling_on_sc=False),
        scratch_types=[
            pltpu.VMEM((n_chunks, chunk), jnp.int32),
            pltpu.VMEM((nbuf, chunk), jnp.int32),
            pltpu.VMEM((nbuf, chunk, dh), jnp.float32),
            pltpu.VMEM_SHARED((n, dh), jnp.float32),
        ] + [pltpu.SemaphoreType.DMA] * (2 * nbuf),
    )
    def seg_kernel(vals_hbm, src_hbm, dst_hbm, out_hbm,
                   src_v, dstr_v, rows_v, acc, *sems):
        gsem = sems[:nbuf]
        dsem = sems[nbuf:]
        cid = lax.axis_index("c")
        sid = lax.axis_index("s")
        half = vals_hbm.at[cid]

        pltpu.async_copy(src_hbm.at[sid], src_v, gsem[0])

        zvec = jnp.zeros((16,), jnp.float32)

        @pl.loop(0, chunk)
        def _(r):
            @pl.loop(0, dh // 16)
            def _(c):
                rows_v.at[0, r, pl.ds(c * 16, 16)][...] = zvec

        @pl.loop(sid, n_row_blocks, step=_NS)
        def _(b):
            pltpu.async_copy(rows_v.at[0], acc.at[pl.ds(b * chunk, chunk)],
                             dsem[0])

        @pl.loop(sid, n_row_blocks, step=_NS)
        def _(b):
            pltpu.make_async_copy(
                rows_v.at[0], acc.at[pl.ds(b * chunk, chunk)], dsem[0]).wait()

        pltpu.make_async_copy(src_hbm.at[sid], src_v, gsem[0]).wait()

        plsc.subcore_barrier()

        for b in range(nbuf):
            pltpu.async_copy(dst_hbm.at[sid, b], dstr_v.at[b], dsem[b])
            pltpu.async_copy(half.at[src_v.at[b]], rows_v.at[b], gsem[b])

        @pl.loop(0, n_chunks - nbuf, step=nbuf)
        def _(i0):
            for b in range(nbuf):
                i = i0 + b
                pltpu.make_async_copy(
                    half.at[src_v.at[i]], rows_v.at[b], gsem[b]).wait()
                pltpu.make_async_copy(
                    dst_hbm.at[sid, i], dstr_v.at[b], dsem[b]).wait()
                pltpu.sync_copy(rows_v.at[b], acc.at[dstr_v.at[b]], add=True)
                pltpu.async_copy(
                    dst_hbm.at[sid, i + nbuf], dstr_v.at[b], dsem[b])
                pltpu.async_copy(
                    half.at[src_v.at[i + nbuf]], rows_v.at[b], gsem[b])

        for b in range(nbuf):
            i = n_chunks - nbuf + b
            pltpu.make_async_copy(
                half.at[src_v.at[i]], rows_v.at[b], gsem[b]).wait()
            pltpu.make_async_copy(
                dst_hbm.at[sid, i], dstr_v.at[b], dsem[b]).wait()
            pltpu.sync_copy(rows_v.at[b], acc.at[dstr_v.at[b]], add=True)

        plsc.subcore_barrier()

        @pl.loop(sid, n_row_blocks, step=_NS)
        def _(b):
            pltpu.async_copy(acc.at[pl.ds(b * chunk, chunk)],
                             out_hbm.at[cid, pl.ds(b * chunk, chunk)],
                             gsem[0])

        @pl.loop(sid, n_row_blocks, step=_NS)
        def _(b):
            pltpu.make_async_copy(
                acc.at[pl.ds(b * chunk, chunk)],
                out_hbm.at[cid, pl.ds(b * chunk, chunk)], gsem[0]).wait()

    return seg_kernel(vals2, src3, dst3)


def _tc_fused_dense(p, w1, w2):
    n = p.shape[1]
    dh = w2.shape[1] // 2

    def body(p_ref, w1_ref, w2_ref, o_ref):
        hidden = jnp.maximum(
            jnp.dot(p_ref[0] + p_ref[1], w1_ref[...],
                    preferred_element_type=jnp.float32,
                    precision=lax.Precision.HIGHEST), 0.0)
        h2 = jnp.dot(hidden, w2_ref[...],
                     preferred_element_type=jnp.float32,
                     precision=lax.Precision.HIGHEST)
        o_ref[0, ...] = h2[:, :dh]
        o_ref[1, ...] = h2[:, dh:]

    return pl.pallas_call(
        body,
        out_shape=jax.ShapeDtypeStruct((2, n, dh), jnp.float32),
    )(p, w1, w2)


def _edge_slabs(edge_index, workers, chunk):
    e = edge_index.shape[1]
    n_chunks = (e // workers) // chunk
    src3 = edge_index[0].reshape(workers, n_chunks, chunk)
    dst3 = edge_index[1].reshape(workers, n_chunks, chunk)
    return src3, dst3


def kernel(x, edge_index, W1, W2):
    x = x.astype(jnp.float32)
    src40, dst40 = _edge_slabs(edge_index, _NW, 40)
    src80, dst80 = _edge_slabs(edge_index, _NS, 80)

    p = _segment_sum_sc(x, src40, dst40, nbuf=5)
    h2 = _tc_fused_dense(p, W1, W2)
    q = _segment_sum_sc_colsplit(h2, src80, dst80, nbuf=5)
    return jnp.concatenate([q[0], q[1]], axis=1)
